# Initial kernel scaffold; baseline (speedup 1.0000x reference)
import jax
import jax.numpy as jnp
from jax.experimental import pallas as pl


def kernel(latents, num_atoms, atom_types, gt_frac_coords, gt_cart_coords, lengths, angles, batch, atom_emb, W_e1, b_e1, W_e2, b_e2, W_m1, b_m1, W_m2, b_m2, W_n1, b_n1, W_n2, b_n2, sigmas):
    raise NotImplementedError("write your pallas kernel here")


# speedup vs baseline: 2.8627x; 2.8627x over previous
import functools

import jax
import jax.numpy as jnp
from jax import lax
from jax.experimental import pallas as pl
from jax.experimental.pallas import tpu as pltpu
from jax.experimental.pallas import tpu_sc as plsc

N = 10000
G = 128
HID = 128
FC = 256
K = 16
NUM_T = 50
NUM_TYPES = 100
CUT2 = 25.0
MIN2 = 1e-12

NPAD = 10240
D_TBL = 128
BR1 = 256
BR2 = 128
BR4 = 128
NEG = -1.0e30


def _prep_kernel(types_ref, batch_ref, cart_ref, noise_ref, ts_ref, sig_ref,
                 lat_tbl_ref, emb_tbl_ref, we1j_ref,
                 emb_out, pert_out, lat_out, cj_out):
    ts = ts_ref[...]
    oh_t = (lax.broadcasted_iota(jnp.int32, (NUM_T, G), 0) == ts
            ).astype(jnp.float32)
    sig_row = jnp.sum(oh_t * sig_ref[...], axis=0, keepdims=True)

    b = batch_ref[...]
    oh_b = (b == lax.broadcasted_iota(jnp.int32, (BR1, G), 1)
            ).astype(jnp.float32)
    sig_n = jnp.sum(oh_b * sig_row, axis=1, keepdims=True)
    lat_out[...] = jnp.dot(oh_b, lat_tbl_ref[...],
                           preferred_element_type=jnp.float32)

    t = types_ref[...]
    oh_ty = (t == lax.broadcasted_iota(jnp.int32, (BR1, NUM_TYPES), 1)
             ).astype(jnp.float32)
    emb = jnp.dot(oh_ty, emb_tbl_ref[...],
                  preferred_element_type=jnp.float32)
    emb_out[...] = emb
    cj_out[...] = jnp.dot(emb, we1j_ref[...],
                          preferred_element_type=jnp.float32)
    pert_out[...] = cart_ref[...] + sig_n * noise_ref[...]


def _nbr_kernel(pert_ref, pertT_ref, batch_ref, batchT_ref,
                jidx_out, vm_out, dx_out, dy_out, dz_out):
    blk = pl.program_id(0)
    pr = pert_ref[...]
    xr = [pertT_ref[c:c + 1, :] for c in range(3)]
    d2 = jnp.zeros((BR2, NPAD), jnp.float32)
    for c in range(3):
        diff = pr[:, c:c + 1] - xr[c]
        d2 = d2 + diff * diff

    col = lax.broadcasted_iota(jnp.int32, (BR2, NPAD), 1)
    row_gid = blk * BR2 + lax.broadcasted_iota(jnp.int32, (BR2, 1), 0)
    same = batch_ref[...] == batchT_ref[...]
    mask = same & (col != row_gid) & (d2 < CUT2) & (d2 > MIN2)
    neg = jnp.where(mask, -d2, NEG)

    js, vs = [], []
    ds = [[], [], []]
    big = jnp.int32(2 ** 30)
    for _ in range(K):
        m = jnp.max(neg, axis=1, keepdims=True)
        hit = neg == m
        idx = jnp.min(jnp.where(hit, col, big), axis=1, keepdims=True)
        ok = m > -1.0e29
        js.append(jnp.where(ok, idx, row_gid))
        vs.append(ok.astype(jnp.float32))
        selm = (col == idx).astype(jnp.float32)
        for c in range(3):
            xj = jnp.sum(selm * xr[c], axis=1, keepdims=True)
            ds[c].append(pr[:, c:c + 1] - xj)
        neg = jnp.where(selm > 0.0, NEG, neg)
    jidx_out[...] = jnp.concatenate(js, axis=1)
    vm_out[...] = jnp.concatenate(vs, axis=1)
    dx_out[...] = jnp.concatenate(ds[0], axis=1)
    dy_out[...] = jnp.concatenate(ds[1], axis=1)
    dz_out[...] = jnp.concatenate(ds[2], axis=1)


def _make_sc_gather():
    info = plsc.get_sparse_core_info()
    nw = info.num_cores * info.num_subcores
    b_per_w = (NPAD * K) // nw
    chunk = 512
    n_chunks = b_per_w // chunk
    mesh = plsc.VectorSubcoreMesh(core_axis_name="c", subcore_axis_name="s")

    @functools.partial(
        pl.kernel, mesh=mesh,
        out_type=jax.ShapeDtypeStruct((NPAD * K, D_TBL), jnp.float32),
        scratch_types=[
            pltpu.VMEM((chunk,), jnp.int32),
            pltpu.VMEM((chunk, D_TBL), jnp.float32),
            pltpu.SemaphoreType.DMA,
        ],
    )
    def gather(table_hbm, idx_hbm, out_hbm, idx_v, rows_v, sem):
        wid = lax.axis_index("s") * info.num_cores + lax.axis_index("c")
        base = wid * b_per_w
        for c in range(n_chunks):
            off = base + c * chunk
            pltpu.sync_copy(idx_hbm.at[pl.ds(off, chunk)], idx_v)
            pltpu.async_copy(table_hbm.at[idx_v], rows_v, sem).wait()
            pltpu.sync_copy(rows_v, out_hbm.at[pl.ds(off, chunk)])

    return gather


def _rep16(a):
    n, c = a.shape
    return jnp.broadcast_to(a[:, None, :], (n, K, c)).reshape(n * K, c)


def _mlp_kernel(cj_ref, emb_ref, pert_ref, lat_ref, vm_ref,
                dx_ref, dy_ref, dz_ref, noise_ref, batchT_ref,
                we1i_ref, we1g_ref, be1_ref,
                we2_ref, be2_ref,
                wm1g_ref, wm1e_ref, wm1l_ref, bm1_ref,
                wm2_ref, bm2_ref,
                wn1p_ref, wn1a_ref, bn1_ref, wn2_ref, bn2_ref,
                acc_se, acc_cnt, loss_out):
    blk = pl.program_id(0)
    nblk = pl.num_programs(0)

    cj = cj_ref[...]
    emb_i = _rep16(emb_ref[...])
    pert_node = pert_ref[...]
    pert_i = _rep16(pert_node)
    lat_e = _rep16(lat_ref[...])
    vm = vm_ref[...]
    dx = dx_ref[...]
    dy = dy_ref[...]
    dz = dz_ref[...]

    dist = jnp.sqrt(dx * dx + dy * dy + dz * dz + 1e-12)

    we1g = we1g_ref[...]
    pre = (cj
           + jnp.dot(emb_i, we1i_ref[...], preferred_element_type=jnp.float32)
           + dx * we1g[0:1, :]
           + dy * we1g[1:2, :]
           + dz * we1g[2:3, :]
           + dist * we1g[3:4, :]
           + be1_ref[...])
    h = jax.nn.silu(pre)
    ee = (jnp.dot(h, we2_ref[...], preferred_element_type=jnp.float32)
          + be2_ref[...]) * vm

    pjx = pert_i[:, 0:1] - dx
    pjy = pert_i[:, 1:2] - dy
    pjz = pert_i[:, 2:3] - dz
    wm1g = wm1g_ref[...]
    prem = (jnp.dot(ee, wm1e_ref[...], preferred_element_type=jnp.float32)
            + jnp.dot(lat_e, wm1l_ref[...], preferred_element_type=jnp.float32)
            + pjx * wm1g[0:1, :]
            + pjy * wm1g[1:2, :]
            + pjz * wm1g[2:3, :]
            + pert_i[:, 0:1] * wm1g[3:4, :]
            + pert_i[:, 1:2] * wm1g[4:5, :]
            + pert_i[:, 2:3] * wm1g[5:6, :]
            + bm1_ref[...])
    m = jax.nn.silu(prem)
    m2 = (jnp.dot(m, wm2_ref[...], preferred_element_type=jnp.float32)
          + bm2_ref[...]) * vm

    agg = jnp.sum(m2.reshape(BR4, K, FC), axis=1)

    wn1p = wn1p_ref[...]
    pren = (jnp.dot(agg, wn1a_ref[...], preferred_element_type=jnp.float32)
            + pert_node[:, 0:1] * wn1p[0:1, :]
            + pert_node[:, 1:2] * wn1p[1:2, :]
            + pert_node[:, 2:3] * wn1p[2:3, :]
            + bn1_ref[...])
    sc = (jnp.dot(jax.nn.silu(pren), wn2_ref[...],
                  preferred_element_type=jnp.float32) + bn2_ref[...])
    se = (sc + noise_ref[...]) ** 2

    bt = batchT_ref[0]
    ohg = (lax.broadcasted_iota(jnp.int32, (G, BR4), 0) == bt
           ).astype(jnp.float32)
    seg = jnp.dot(ohg, se, preferred_element_type=jnp.float32)
    cnt = jnp.sum(ohg, axis=1, keepdims=True)

    @pl.when(blk == 0)
    def _():
        acc_se[...] = jnp.zeros_like(acc_se)
        acc_cnt[...] = jnp.zeros_like(acc_cnt)

    acc_se[...] += seg
    acc_cnt[...] += cnt

    @pl.when(blk == nblk - 1)
    def _():
        per = acc_se[...] / jnp.maximum(acc_cnt[...], 1.0)
        tot = jnp.sum(jnp.sum(per, axis=0, keepdims=True),
                      axis=1, keepdims=True)
        loss_out[...] = tot / (G * 3)


def kernel(latents, num_atoms, atom_types, gt_frac_coords, gt_cart_coords,
           lengths, angles, batch, atom_emb, W_e1, b_e1, W_e2, b_e2,
           W_m1, b_m1, W_m2, b_m2, W_n1, b_n1, W_n2, b_n2, sigmas):
    f32 = jnp.float32
    time_steps = jax.random.randint(jax.random.key(1), (G,), 0, NUM_T)
    noise = jax.random.normal(jax.random.key(2), (N, 3), f32)

    pad = NPAD - N
    types_p = jnp.concatenate([atom_types.astype(jnp.int32),
                               jnp.zeros((pad,), jnp.int32)]).reshape(NPAD, 1)
    batch_p = jnp.concatenate([batch.astype(jnp.int32),
                               jnp.full((pad,), -1, jnp.int32)]).reshape(NPAD, 1)
    cart_p = jnp.concatenate([gt_cart_coords, jnp.zeros((pad, 3), f32)])
    noise_p = jnp.concatenate([noise, jnp.zeros((pad, 3), f32)])

    grid1 = NPAD // BR1
    emb_n, pert, lat_n, cj_tbl = pl.pallas_call(
        _prep_kernel,
        grid=(grid1,),
        in_specs=[
            pl.BlockSpec((BR1, 1), lambda i: (i, 0)),
            pl.BlockSpec((BR1, 1), lambda i: (i, 0)),
            pl.BlockSpec((BR1, 3), lambda i: (i, 0)),
            pl.BlockSpec((BR1, 3), lambda i: (i, 0)),
            pl.BlockSpec((1, G), lambda i: (0, 0)),
            pl.BlockSpec((NUM_T, 1), lambda i: (0, 0)),
            pl.BlockSpec((G, HID), lambda i: (0, 0)),
            pl.BlockSpec((NUM_TYPES, HID), lambda i: (0, 0)),
            pl.BlockSpec((HID, HID), lambda i: (0, 0)),
        ],
        out_specs=[
            pl.BlockSpec((BR1, HID), lambda i: (i, 0)),
            pl.BlockSpec((BR1, 3), lambda i: (i, 0)),
            pl.BlockSpec((BR1, HID), lambda i: (i, 0)),
            pl.BlockSpec((BR1, HID), lambda i: (i, 0)),
        ],
        out_shape=[
            jax.ShapeDtypeStruct((NPAD, HID), f32),
            jax.ShapeDtypeStruct((NPAD, 3), f32),
            jax.ShapeDtypeStruct((NPAD, HID), f32),
            jax.ShapeDtypeStruct((NPAD, HID), f32),
        ],
    )(types_p, batch_p, cart_p, noise_p,
      time_steps.astype(jnp.int32).reshape(1, G), sigmas.reshape(NUM_T, 1),
      latents, atom_emb, W_e1[:HID])

    grid2 = NPAD // BR2
    j_idx, vm, ddx, ddy, ddz = pl.pallas_call(
        _nbr_kernel,
        grid=(grid2,),
        in_specs=[
            pl.BlockSpec((BR2, 3), lambda i: (i, 0)),
            pl.BlockSpec((3, NPAD), lambda i: (0, 0)),
            pl.BlockSpec((BR2, 1), lambda i: (i, 0)),
            pl.BlockSpec((1, NPAD), lambda i: (0, 0)),
        ],
        out_specs=[
            pl.BlockSpec((BR2, K), lambda i: (i, 0)),
            pl.BlockSpec((BR2, K), lambda i: (i, 0)),
            pl.BlockSpec((BR2, K), lambda i: (i, 0)),
            pl.BlockSpec((BR2, K), lambda i: (i, 0)),
            pl.BlockSpec((BR2, K), lambda i: (i, 0)),
        ],
        out_shape=[
            jax.ShapeDtypeStruct((NPAD, K), jnp.int32),
            jax.ShapeDtypeStruct((NPAD, K), f32),
            jax.ShapeDtypeStruct((NPAD, K), f32),
            jax.ShapeDtypeStruct((NPAD, K), f32),
            jax.ShapeDtypeStruct((NPAD, K), f32),
        ],
    )(pert, pert.T, batch_p, batch_p.reshape(1, NPAD))

    gathered = _make_sc_gather()(cj_tbl, j_idx.reshape(-1))

    grid4 = NPAD // BR4
    acc_se, acc_cnt, loss = pl.pallas_call(
        _mlp_kernel,
        grid=(grid4,),
        in_specs=[
            pl.BlockSpec((BR4 * K, D_TBL), lambda i: (i, 0)),
            pl.BlockSpec((BR4, HID), lambda i: (i, 0)),
            pl.BlockSpec((BR4, 3), lambda i: (i, 0)),
            pl.BlockSpec((BR4, HID), lambda i: (i, 0)),
            pl.BlockSpec((BR4 * K, 1), lambda i: (i, 0)),
            pl.BlockSpec((BR4 * K, 1), lambda i: (i, 0)),
            pl.BlockSpec((BR4 * K, 1), lambda i: (i, 0)),
            pl.BlockSpec((BR4 * K, 1), lambda i: (i, 0)),
            pl.BlockSpec((BR4, 3), lambda i: (i, 0)),
            pl.BlockSpec((1, 1, BR4), lambda i: (i, 0, 0)),
            pl.BlockSpec((HID, HID), lambda i: (0, 0)),
            pl.BlockSpec((4, HID), lambda i: (0, 0)),
            pl.BlockSpec((1, HID), lambda i: (0, 0)),
            pl.BlockSpec((HID, HID), lambda i: (0, 0)),
            pl.BlockSpec((1, HID), lambda i: (0, 0)),
            pl.BlockSpec((6, FC), lambda i: (0, 0)),
            pl.BlockSpec((HID, FC), lambda i: (0, 0)),
            pl.BlockSpec((HID, FC), lambda i: (0, 0)),
            pl.BlockSpec((1, FC), lambda i: (0, 0)),
            pl.BlockSpec((FC, FC), lambda i: (0, 0)),
            pl.BlockSpec((1, FC), lambda i: (0, 0)),
            pl.BlockSpec((3, FC), lambda i: (0, 0)),
            pl.BlockSpec((FC, FC), lambda i: (0, 0)),
            pl.BlockSpec((1, FC), lambda i: (0, 0)),
            pl.BlockSpec((FC, 3), lambda i: (0, 0)),
            pl.BlockSpec((1, 3), lambda i: (0, 0)),
        ],
        out_specs=[
            pl.BlockSpec((G, 3), lambda i: (0, 0)),
            pl.BlockSpec((G, 1), lambda i: (0, 0)),
            pl.BlockSpec((1, 1), lambda i: (0, 0)),
        ],
        out_shape=[
            jax.ShapeDtypeStruct((G, 3), f32),
            jax.ShapeDtypeStruct((G, 1), f32),
            jax.ShapeDtypeStruct((1, 1), f32),
        ],
    )(gathered, emb_n, pert, lat_n,
      vm.reshape(NPAD * K, 1), ddx.reshape(NPAD * K, 1),
      ddy.reshape(NPAD * K, 1), ddz.reshape(NPAD * K, 1), noise_p,
      batch_p.reshape(grid4, 1, BR4),
      W_e1[HID:2 * HID], W_e1[2 * HID:], b_e1.reshape(1, HID),
      W_e2, b_e2.reshape(1, HID),
      W_m1[:6], W_m1[6:6 + HID], W_m1[6 + HID:], b_m1.reshape(1, FC),
      W_m2, b_m2.reshape(1, FC),
      W_n1[:3], W_n1[3:], b_n1.reshape(1, FC),
      W_n2, b_n2.reshape(1, 3))

    return (loss.reshape(()), jnp.array(0.0, f32), jnp.array(0.0, f32))
